# baseline (speedup 1.0000x reference)
import functools

import jax
import jax.numpy as jnp
from jax import lax
from jax.experimental import pallas as pl
from jax.experimental.pallas import tpu as pltpu
from jax.experimental.pallas import tpu_sc as plsc

N_NODES = 10000
N_EDGES = 320000
HIDDEN = 128
NUM_RBF = 16
CUTOFF_UPPER = 5.0

NUM_CORES = 2
NUM_SUBCORES = 16
NW = NUM_CORES * NUM_SUBCORES
CH = 80
NPAD = 10240
ROWS_PER_TILE = NPAD // NUM_SUBCORES

EBLK = 1280
NBLK = 2000

SPLITS = (30720, 58880, 58880, 58880, 56320, 56320)


def _tanh(x):
    return 1.0 - 2.0 / (jnp.exp(2.0 * x) + 1.0)


def _filter_body(ea_ref, ew_ref, w1_ref, b1_ref, w2_ref, b2_ref, out_ref):
    t = _tanh(lax.dot_general(ea_ref[...], w1_ref[...],
                              (((0,), (0,)), ((), ())),
                              preferred_element_type=jnp.float32) + b1_ref[...])
    w = jnp.dot(t, w2_ref[...], preferred_element_type=jnp.float32) + b2_ref[...]
    c = 0.5 * (jnp.cos(ew_ref[...] * (jnp.pi / CUTOFF_UPPER)) + 1.0)
    out_ref[...] = w * c.reshape(EBLK, 1)


def _filter_w(edge_attr, edge_weight3d, w1, b1, w2, b2, nblk, blk_off):
    return pl.pallas_call(
        _filter_body,
        grid=(nblk,),
        in_specs=[
            pl.BlockSpec((NUM_RBF, EBLK), lambda i: (0, i + blk_off)),
            pl.BlockSpec((1, 1, EBLK), lambda i: (i + blk_off, 0, 0)),
            pl.BlockSpec((NUM_RBF, HIDDEN), lambda i: (0, 0)),
            pl.BlockSpec((1, HIDDEN), lambda i: (0, 0)),
            pl.BlockSpec((HIDDEN, HIDDEN), lambda i: (0, 0)),
            pl.BlockSpec((1, HIDDEN), lambda i: (0, 0)),
        ],
        out_specs=pl.BlockSpec((EBLK, HIDDEN), lambda i: (i, 0)),
        out_shape=jax.ShapeDtypeStruct((nblk * EBLK, HIDDEN), jnp.float32),
    )(edge_attr, edge_weight3d, w1, b1, w2, b2)


def _proj_body(x_ref, w_ref, out_ref):
    out_ref[...] = jnp.dot(x_ref[...], w_ref[...],
                           preferred_element_type=jnp.float32)


def _proj(x, lin1_w):
    grid = (N_NODES // NBLK,)
    return pl.pallas_call(
        _proj_body,
        grid=grid,
        in_specs=[
            pl.BlockSpec((NBLK, HIDDEN), lambda i: (i, 0)),
            pl.BlockSpec((HIDDEN, HIDDEN), lambda i: (0, 0)),
        ],
        out_specs=pl.BlockSpec((NBLK, HIDDEN), lambda i: (i, 0)),
        out_shape=jax.ShapeDtypeStruct((N_NODES, HIDDEN), jnp.float32),
    )(x, lin1_w)


def _make_conv_body(epw, nchunk, eoff):
  def _conv_body(h_hbm, w_hbm, src_hbm, dst_hbm, init_hbm, out_hbm,
                 src_v0, src_v1, dst_v0, dst_v1, dst_s0, dst_s1,
                 rows_v0, rows_v1, wv0, wv1,
                 agg_sh, sem_i0, sem_i1, sem_g0, sem_g1, sem_w0, sem_w1,
                 sem_s0, sem_s1):
    cid = lax.axis_index("c")
    sid = lax.axis_index("s")
    wid = sid * NUM_CORES + cid
    row0 = sid * ROWS_PER_TILE
    e0 = eoff + wid * epw
    l0 = wid * epw
    src_v = (src_v0, src_v1)
    dst_v = (dst_v0, dst_v1)
    dst_s = (dst_s0, dst_s1)
    rows_v = (rows_v0, rows_v1)
    wv = (wv0, wv1)
    sem_i = (sem_i0, sem_i1)
    sem_g = (sem_g0, sem_g1)
    sem_w = (sem_w0, sem_w1)
    sem_s = (sem_s0, sem_s1)

    def idx_copy(j, b):
        base = e0 + j * CH
        return (pltpu.make_async_copy(src_hbm.at[pl.ds(base, CH)], src_v[b],
                                      sem_i[b]),
                pltpu.make_async_copy(dst_hbm.at[pl.ds(base, CH)], dst_v[b],
                                      sem_i[b]))

    def gather(b):
        return pltpu.make_async_copy(h_hbm.at[src_v[b]], rows_v[b], sem_g[b])

    def w_copy(j, b):
        base = l0 + j * CH
        return pltpu.make_async_copy(w_hbm.at[pl.ds(base, CH)], wv[b],
                                     sem_w[b])

    def scat_start(b):
        pltpu.async_copy(rows_v[b], agg_sh.at[dst_s[b]], sem_s[b], add=True)

    def scat_wait(b):
        pltpu.make_async_copy(rows_v[b], agg_sh.at[dst_s[b]], sem_s[b]).wait()

    def save_dst(b):
        for c in range(CH // 16):
            s = pl.ds(c * 16, 16)
            dst_s[b][s] = dst_v[b][s]

    def mul(b):
        def mbody(e, c2):
            for u in range(2):
                ee = e * 2 + u
                for g in range(HIDDEN // 16):
                    s = pl.ds(g * 16, 16)
                    rows_v[b][ee, s] = rows_v[b][ee, s] * wv[b][ee, s]
            return c2
        lax.fori_loop(0, CH // 2, mbody, 0)

    pltpu.sync_copy(init_hbm.at[cid, pl.ds(row0, ROWS_PER_TILE)],
                    agg_sh.at[pl.ds(row0, ROWS_PER_TILE)])
    plsc.subcore_barrier()

    a, c = idx_copy(0, 0)
    a.start(); c.start(); a.wait(); c.wait()
    gather(0).start()
    w_copy(0, 0).start()
    a, c = idx_copy(1, 1)
    a.start(); c.start()

    def step(j, b):
        nb = 1 - b
        gather(b).wait()
        w_copy(j, b).wait()
        save_dst(b)

        @pl.when(j + 1 < nchunk)
        def _():
            a2, c2 = idx_copy(j + 1, nb)
            a2.wait(); c2.wait()

            @pl.when(j >= 1)
            def _():
                scat_wait(nb)
            gather(nb).start()
            w_copy(j + 1, nb).start()

        @pl.when(j + 2 < nchunk)
        def _():
            a3, c3 = idx_copy(j + 2, b)
            a3.start(); c3.start()

        mul(b)
        scat_start(b)

    def pair(k, carry):
        step(2 * k, 0)
        step(2 * k + 1, 1)
        return carry

    lax.fori_loop(0, nchunk // 2, pair, 0)
    if nchunk % 2 == 1:
        step(nchunk - 1, 0)
    scat_wait(1 - (nchunk - 1) % 2)
    scat_wait((nchunk - 1) % 2)

    plsc.subcore_barrier()
    pltpu.sync_copy(agg_sh.at[pl.ds(row0, ROWS_PER_TILE)],
                    out_hbm.at[cid, pl.ds(row0, ROWS_PER_TILE)])

  return _conv_body


def _conv_sc(h, w, src, dst, init, epw, nchunk, eoff):
    mesh = plsc.VectorSubcoreMesh(core_axis_name="c", subcore_axis_name="s")
    f = functools.partial(
        pl.kernel,
        mesh=mesh,
        out_type=jax.ShapeDtypeStruct((NUM_CORES, NPAD, HIDDEN), jnp.float32),
        scratch_types=[
            pltpu.VMEM((CH,), jnp.int32),
            pltpu.VMEM((CH,), jnp.int32),
            pltpu.VMEM((CH,), jnp.int32),
            pltpu.VMEM((CH,), jnp.int32),
            pltpu.VMEM((CH,), jnp.int32),
            pltpu.VMEM((CH,), jnp.int32),
            pltpu.VMEM((CH, HIDDEN), jnp.float32),
            pltpu.VMEM((CH, HIDDEN), jnp.float32),
            pltpu.VMEM((CH, HIDDEN), jnp.float32),
            pltpu.VMEM((CH, HIDDEN), jnp.float32),
            pltpu.VMEM_SHARED((NPAD, HIDDEN), jnp.float32),
            pltpu.SemaphoreType.DMA,
            pltpu.SemaphoreType.DMA,
            pltpu.SemaphoreType.DMA,
            pltpu.SemaphoreType.DMA,
            pltpu.SemaphoreType.DMA,
            pltpu.SemaphoreType.DMA,
            pltpu.SemaphoreType.DMA,
            pltpu.SemaphoreType.DMA,
        ],
    )(_make_conv_body(epw, nchunk, eoff))
    return f(h, w, src, dst, init)


def _epilogue_body(p0_ref, p1_ref, w2_ref, b2_ref, w3_ref, b3_ref, out_ref):
    agg = p0_ref[...] + p1_ref[...]
    y = jnp.dot(agg, w2_ref[...], preferred_element_type=jnp.float32) + b2_ref[...]
    out_ref[...] = jnp.dot(_tanh(y), w3_ref[...],
                           preferred_element_type=jnp.float32) + b3_ref[...]


def _epilogue(p0, p1, lin2_w, lin2_b2d, lin_w, lin_b2d):
    grid = (N_NODES // NBLK,)
    return pl.pallas_call(
        _epilogue_body,
        grid=grid,
        in_specs=[
            pl.BlockSpec((NBLK, HIDDEN), lambda i: (i, 0)),
            pl.BlockSpec((NBLK, HIDDEN), lambda i: (i, 0)),
            pl.BlockSpec((HIDDEN, HIDDEN), lambda i: (0, 0)),
            pl.BlockSpec((1, HIDDEN), lambda i: (0, 0)),
            pl.BlockSpec((HIDDEN, HIDDEN), lambda i: (0, 0)),
            pl.BlockSpec((1, HIDDEN), lambda i: (0, 0)),
        ],
        out_specs=pl.BlockSpec((NBLK, HIDDEN), lambda i: (i, 0)),
        out_shape=jax.ShapeDtypeStruct((N_NODES, HIDDEN), jnp.float32),
    )(p0, p1, lin2_w, lin2_b2d, lin_w, lin_b2d)


def kernel(x, edge_index, edge_weight, edge_attr, filter_w1, filter_b1,
           filter_w2, filter_b2, lin1_w, lin2_w, lin2_b, lin_w, lin_b):
    src = edge_index[0].astype(jnp.int32)
    dst = edge_index[1].astype(jnp.int32)
    ew3 = edge_weight.reshape(N_EDGES // EBLK, 1, EBLK)
    ea_t = edge_attr.T
    b1 = filter_b1.reshape(1, HIDDEN)
    b2 = filter_b2.reshape(1, HIDDEN)
    h = _proj(x, lin1_w)
    ws = []
    eoff = 0
    for seg in SPLITS:
        ws.append(_filter_w(ea_t, ew3, filter_w1, b1, filter_w2, b2,
                            seg // EBLK, eoff // EBLK))
        eoff += seg
    part = jnp.zeros((NUM_CORES, NPAD, HIDDEN), jnp.float32)
    eoff = 0
    for seg, w_seg in zip(SPLITS, ws):
        part = _conv_sc(h, w_seg, src, dst, part, seg // NW,
                        seg // NW // CH, eoff)
        eoff += seg
    out = _epilogue(part[0], part[1], lin2_w,
                    lin2_b.reshape(1, HIDDEN), lin_w, lin_b.reshape(1, HIDDEN))
    return out


# speedup vs baseline: 1.0373x; 1.0373x over previous
import functools

import jax
import jax.numpy as jnp
from jax import lax
from jax.experimental import pallas as pl
from jax.experimental.pallas import tpu as pltpu
from jax.experimental.pallas import tpu_sc as plsc

N_NODES = 10000
N_EDGES = 320000
HIDDEN = 128
NUM_RBF = 16
CUTOFF_UPPER = 5.0

NUM_CORES = 2
NUM_SUBCORES = 16
NW = NUM_CORES * NUM_SUBCORES
CH = 80
NPAD = 10240
ROWS_PER_TILE = NPAD // NUM_SUBCORES

EBLK = 1280
NBLK = 2000

SPLITS = (79360, 79360, 79360, 81920)


def _tanh(x):
    return 1.0 - 2.0 / (jnp.exp(2.0 * x) + 1.0)


def _filter_body(ea_ref, ew_ref, w1_ref, b1_ref, w2_ref, b2_ref, out_ref):
    t = _tanh(lax.dot_general(ea_ref[...], w1_ref[...],
                              (((0,), (0,)), ((), ())),
                              preferred_element_type=jnp.float32) + b1_ref[...])
    w = jnp.dot(t, w2_ref[...], preferred_element_type=jnp.float32) + b2_ref[...]
    c = 0.5 * (jnp.cos(ew_ref[...] * (jnp.pi / CUTOFF_UPPER)) + 1.0)
    out_ref[...] = w * c.reshape(EBLK, 1)


def _filter_w(edge_attr, edge_weight3d, w1, b1, w2, b2, nblk, blk_off):
    return pl.pallas_call(
        _filter_body,
        grid=(nblk,),
        in_specs=[
            pl.BlockSpec((NUM_RBF, EBLK), lambda i: (0, i + blk_off)),
            pl.BlockSpec((1, 1, EBLK), lambda i: (i + blk_off, 0, 0)),
            pl.BlockSpec((NUM_RBF, HIDDEN), lambda i: (0, 0)),
            pl.BlockSpec((1, HIDDEN), lambda i: (0, 0)),
            pl.BlockSpec((HIDDEN, HIDDEN), lambda i: (0, 0)),
            pl.BlockSpec((1, HIDDEN), lambda i: (0, 0)),
        ],
        out_specs=pl.BlockSpec((EBLK, HIDDEN), lambda i: (i, 0)),
        out_shape=jax.ShapeDtypeStruct((nblk * EBLK, HIDDEN), jnp.float32),
    )(edge_attr, edge_weight3d, w1, b1, w2, b2)


def _proj_body(x_ref, w_ref, out_ref):
    out_ref[...] = jnp.dot(x_ref[...], w_ref[...],
                           preferred_element_type=jnp.float32)


def _proj(x, lin1_w):
    grid = (N_NODES // NBLK,)
    return pl.pallas_call(
        _proj_body,
        grid=grid,
        in_specs=[
            pl.BlockSpec((NBLK, HIDDEN), lambda i: (i, 0)),
            pl.BlockSpec((HIDDEN, HIDDEN), lambda i: (0, 0)),
        ],
        out_specs=pl.BlockSpec((NBLK, HIDDEN), lambda i: (i, 0)),
        out_shape=jax.ShapeDtypeStruct((N_NODES, HIDDEN), jnp.float32),
    )(x, lin1_w)


def _make_conv_body(epw, nchunk, eoff):
  def _conv_body(h_hbm, w_hbm, src_hbm, dst_hbm, init_hbm, out_hbm,
                 src_v0, src_v1, dst_v0, dst_v1, dst_s0, dst_s1,
                 rows_v0, rows_v1, wv0, wv1,
                 agg_sh, sem_i0, sem_i1, sem_g0, sem_g1, sem_w0, sem_w1,
                 sem_s0, sem_s1):
    cid = lax.axis_index("c")
    sid = lax.axis_index("s")
    wid = sid * NUM_CORES + cid
    row0 = sid * ROWS_PER_TILE
    e0 = eoff + wid * epw
    l0 = wid * epw
    src_v = (src_v0, src_v1)
    dst_v = (dst_v0, dst_v1)
    dst_s = (dst_s0, dst_s1)
    rows_v = (rows_v0, rows_v1)
    wv = (wv0, wv1)
    sem_i = (sem_i0, sem_i1)
    sem_g = (sem_g0, sem_g1)
    sem_w = (sem_w0, sem_w1)
    sem_s = (sem_s0, sem_s1)

    def idx_copy(j, b):
        base = e0 + j * CH
        return (pltpu.make_async_copy(src_hbm.at[pl.ds(base, CH)], src_v[b],
                                      sem_i[b]),
                pltpu.make_async_copy(dst_hbm.at[pl.ds(base, CH)], dst_v[b],
                                      sem_i[b]))

    def gather(b):
        return pltpu.make_async_copy(h_hbm.at[src_v[b]], rows_v[b], sem_g[b])

    def w_copy(j, b):
        base = l0 + j * CH
        return pltpu.make_async_copy(w_hbm.at[pl.ds(base, CH)], wv[b],
                                     sem_w[b])

    def scat_start(b):
        pltpu.async_copy(rows_v[b], agg_sh.at[dst_s[b]], sem_s[b], add=True)

    def scat_wait(b):
        pltpu.make_async_copy(rows_v[b], agg_sh.at[dst_s[b]], sem_s[b]).wait()

    def save_dst(b):
        for c in range(CH // 16):
            s = pl.ds(c * 16, 16)
            dst_s[b][s] = dst_v[b][s]

    def mul(b):
        def mbody(e, c2):
            for u in range(2):
                ee = e * 2 + u
                for g in range(HIDDEN // 16):
                    s = pl.ds(g * 16, 16)
                    rows_v[b][ee, s] = rows_v[b][ee, s] * wv[b][ee, s]
            return c2
        lax.fori_loop(0, CH // 2, mbody, 0)

    pltpu.sync_copy(init_hbm.at[cid, pl.ds(row0, ROWS_PER_TILE)],
                    agg_sh.at[pl.ds(row0, ROWS_PER_TILE)])
    plsc.subcore_barrier()

    a, c = idx_copy(0, 0)
    a.start(); c.start(); a.wait(); c.wait()
    gather(0).start()
    w_copy(0, 0).start()
    a, c = idx_copy(1, 1)
    a.start(); c.start()

    def step(j, b):
        nb = 1 - b
        gather(b).wait()
        w_copy(j, b).wait()
        save_dst(b)

        @pl.when(j + 1 < nchunk)
        def _():
            a2, c2 = idx_copy(j + 1, nb)
            a2.wait(); c2.wait()

            @pl.when(j >= 1)
            def _():
                scat_wait(nb)
            gather(nb).start()
            w_copy(j + 1, nb).start()

        @pl.when(j + 2 < nchunk)
        def _():
            a3, c3 = idx_copy(j + 2, b)
            a3.start(); c3.start()

        mul(b)
        scat_start(b)

    def pair(k, carry):
        step(2 * k, 0)
        step(2 * k + 1, 1)
        return carry

    lax.fori_loop(0, nchunk // 2, pair, 0)
    if nchunk % 2 == 1:
        step(nchunk - 1, 0)
    scat_wait(1 - (nchunk - 1) % 2)
    scat_wait((nchunk - 1) % 2)

    plsc.subcore_barrier()
    pltpu.sync_copy(agg_sh.at[pl.ds(row0, ROWS_PER_TILE)],
                    out_hbm.at[cid, pl.ds(row0, ROWS_PER_TILE)])

  return _conv_body


def _conv_sc(h, w, src, dst, init, epw, nchunk, eoff):
    mesh = plsc.VectorSubcoreMesh(core_axis_name="c", subcore_axis_name="s")
    f = functools.partial(
        pl.kernel,
        mesh=mesh,
        out_type=jax.ShapeDtypeStruct((NUM_CORES, NPAD, HIDDEN), jnp.float32),
        scratch_types=[
            pltpu.VMEM((CH,), jnp.int32),
            pltpu.VMEM((CH,), jnp.int32),
            pltpu.VMEM((CH,), jnp.int32),
            pltpu.VMEM((CH,), jnp.int32),
            pltpu.VMEM((CH,), jnp.int32),
            pltpu.VMEM((CH,), jnp.int32),
            pltpu.VMEM((CH, HIDDEN), jnp.float32),
            pltpu.VMEM((CH, HIDDEN), jnp.float32),
            pltpu.VMEM((CH, HIDDEN), jnp.float32),
            pltpu.VMEM((CH, HIDDEN), jnp.float32),
            pltpu.VMEM_SHARED((NPAD, HIDDEN), jnp.float32),
            pltpu.SemaphoreType.DMA,
            pltpu.SemaphoreType.DMA,
            pltpu.SemaphoreType.DMA,
            pltpu.SemaphoreType.DMA,
            pltpu.SemaphoreType.DMA,
            pltpu.SemaphoreType.DMA,
            pltpu.SemaphoreType.DMA,
            pltpu.SemaphoreType.DMA,
        ],
    )(_make_conv_body(epw, nchunk, eoff))
    return f(h, w, src, dst, init)


def _epilogue_body(p0_ref, p1_ref, w2_ref, b2_ref, w3_ref, b3_ref, out_ref):
    agg = p0_ref[...] + p1_ref[...]
    y = jnp.dot(agg, w2_ref[...], preferred_element_type=jnp.float32) + b2_ref[...]
    out_ref[...] = jnp.dot(_tanh(y), w3_ref[...],
                           preferred_element_type=jnp.float32) + b3_ref[...]


def _epilogue(p0, p1, lin2_w, lin2_b2d, lin_w, lin_b2d):
    grid = (N_NODES // NBLK,)
    return pl.pallas_call(
        _epilogue_body,
        grid=grid,
        in_specs=[
            pl.BlockSpec((NBLK, HIDDEN), lambda i: (i, 0)),
            pl.BlockSpec((NBLK, HIDDEN), lambda i: (i, 0)),
            pl.BlockSpec((HIDDEN, HIDDEN), lambda i: (0, 0)),
            pl.BlockSpec((1, HIDDEN), lambda i: (0, 0)),
            pl.BlockSpec((HIDDEN, HIDDEN), lambda i: (0, 0)),
            pl.BlockSpec((1, HIDDEN), lambda i: (0, 0)),
        ],
        out_specs=pl.BlockSpec((NBLK, HIDDEN), lambda i: (i, 0)),
        out_shape=jax.ShapeDtypeStruct((N_NODES, HIDDEN), jnp.float32),
    )(p0, p1, lin2_w, lin2_b2d, lin_w, lin_b2d)


def kernel(x, edge_index, edge_weight, edge_attr, filter_w1, filter_b1,
           filter_w2, filter_b2, lin1_w, lin2_w, lin2_b, lin_w, lin_b):
    src = edge_index[0].astype(jnp.int32)
    dst = edge_index[1].astype(jnp.int32)
    ew3 = edge_weight.reshape(N_EDGES // EBLK, 1, EBLK)
    ea_t = edge_attr.T
    b1 = filter_b1.reshape(1, HIDDEN)
    b2 = filter_b2.reshape(1, HIDDEN)
    h = _proj(x, lin1_w)
    ws = []
    eoff = 0
    for seg in SPLITS:
        ws.append(_filter_w(ea_t, ew3, filter_w1, b1, filter_w2, b2,
                            seg // EBLK, eoff // EBLK))
        eoff += seg
    part = jnp.zeros((NUM_CORES, NPAD, HIDDEN), jnp.float32)
    eoff = 0
    for seg, w_seg in zip(SPLITS, ws):
        part = _conv_sc(h, w_seg, src, dst, part, seg // NW,
                        seg // NW // CH, eoff)
        eoff += seg
    out = _epilogue(part[0], part[1], lin2_w,
                    lin2_b.reshape(1, HIDDEN), lin_w, lin_b.reshape(1, HIDDEN))
    return out
